# baseline (speedup 1.0000x reference)
import functools
import math

import jax
import jax.numpy as jnp
from jax import lax
from jax.experimental import pallas as pl
from jax.experimental.pallas import tpu as pltpu
from jax.experimental.pallas import tpu_sc as plsc

NC = 2
NS = 16
GG = 1024

EB = 32
CH = 80

_SC_PARAMS = pltpu.CompilerParams(
    needs_layout_passes=False, use_tc_tiling_on_sc=False)


def _edge_compute(C, lo, hi, qrows, krows, vrows, dstv, idxv, stage,
                  denstage, SCALE):
  iota16 = lax.iota(jnp.int32, 16)

  def group(g, carry):
    s0 = g * 16
    rows = iota16 + s0
    accs = [jnp.zeros((16,), jnp.float32) for _ in range(4)]
    col = jnp.zeros((16,), jnp.int32)
    for u in range(C):
      accs[u % 4] += (plsc.load_gather(qrows, [rows, col]) *
                      plsc.load_gather(krows, [rows, col]))
      col = col + 1
    a = (accs[0] + accs[1]) + (accs[2] + accs[3])
    d16 = dstv[pl.ds(s0, 16)]
    m = (d16 >= lo) & (d16 < hi)
    e16 = jnp.where(m, jnp.exp(a * SCALE), 0.0)
    denstage[pl.ds(s0, 16)] = e16
    idxv[pl.ds(s0, 16)] = jnp.where(m, d16 - lo, d16 & 32767)
    col = jnp.zeros((16,), jnp.int32)
    for _ in range(C):
      vc = plsc.load_gather(vrows, [rows, col])
      plsc.store_scatter(stage, [rows, col], e16 * vc)
      col = col + 1
    return carry
  lax.fori_loop(0, EB // 16, group, 0)


def _edge_loop(C, lo, hi, NCHUNK, tbase, qt_ref, kt_ref, vt_ref,
               src_ref, dst_ref, bufs, acc, den, SCALE):
  (srcvA, dstvA, srcvB, dstvB, qrA, krA, vrA, qrB, krB, vrB,
   idxvA, idxvB, stageA, stageB, denstageA, denstageB,
   gsemA, gsemB, idsemA, idsemB, ssemA, ssemB) = bufs

  def issue_ids(base, srcv, dstv, sem):
    pltpu.async_copy(src_ref.at[pl.ds(base, EB)], srcv, sem)
    pltpu.async_copy(dst_ref.at[pl.ds(base, EB)], dstv, sem)

  def wait_ids(srcv, dstv, sem):
    pltpu.make_async_copy(src_ref.at[pl.ds(0, EB)], srcv, sem).wait()
    pltpu.make_async_copy(dst_ref.at[pl.ds(0, EB)], dstv, sem).wait()

  def issue_g(srcv, dstv, qr, kr, vr, sem):
    pltpu.async_copy(qt_ref.at[dstv], qr, sem)
    pltpu.async_copy(kt_ref.at[srcv], kr, sem)
    pltpu.async_copy(vt_ref.at[srcv], vr, sem)

  def wait_g(srcv, dstv, qr, kr, vr, sem):
    pltpu.make_async_copy(qt_ref.at[dstv], qr, sem).wait()
    pltpu.make_async_copy(kt_ref.at[srcv], kr, sem).wait()
    pltpu.make_async_copy(vt_ref.at[srcv], vr, sem).wait()

  def compute(srcv, dstv, qr, kr, vr, idxv, stage, denstage, ssem):
    _edge_compute(C, lo, hi, qr, kr, vr, dstv, idxv, stage, denstage, SCALE)
    pltpu.sync_copy(stage, acc.at[idxv], add=True)
    pltpu.sync_copy(denstage, den.at[idxv], add=True)

  def wait_s(idxv, stage, ssem):
    pass

  A = (srcvA, dstvA, qrA, krA, vrA, gsemA)
  B = (srcvB, dstvB, qrB, krB, vrB, gsemB)
  CA = (srcvA, dstvA, qrA, krA, vrA, idxvA, stageA, denstageA, ssemA)
  CB = (srcvB, dstvB, qrB, krB, vrB, idxvB, stageB, denstageB, ssemB)

  issue_ids(tbase, srcvA, dstvA, idsemA)
  wait_ids(srcvA, dstvA, idsemA)
  issue_g(*A)
  issue_ids(tbase + EB, srcvB, dstvB, idsemB)
  wait_ids(srcvB, dstvB, idsemB)
  wait_g(*A)
  issue_g(*B)
  compute(*CA)
  issue_ids(tbase + jnp.minimum(2, NCHUNK - 1) * EB, srcvA, dstvA, idsemA)
  wait_g(*B)
  compute(*CB)
  wait_ids(srcvA, dstvA, idsemA)
  issue_g(*A)
  issue_ids(tbase + jnp.minimum(3, NCHUNK - 1) * EB, srcvB, dstvB, idsemB)

  def iter_j(jj, carry):
    j = jj + 1
    a = 2 * j
    b = a + 1
    wait_ids(srcvB, dstvB, idsemB)
    wait_g(*A)
    issue_g(*B)
    wait_s(idxvA, stageA, ssemA)
    compute(*CA)
    nxt = jnp.minimum(a + 2, NCHUNK - 1)
    issue_ids(tbase + nxt * EB, srcvA, dstvA, idsemA)
    wait_g(*B)
    wait_s(idxvB, stageB, ssemB)
    compute(*CB)
    wait_ids(srcvA, dstvA, idsemA)
    issue_g(*A)
    nxt2 = jnp.minimum(b + 2, NCHUNK - 1)
    issue_ids(tbase + nxt2 * EB, srcvB, dstvB, idsemB)
    return carry
  lax.fori_loop(0, NCHUNK // 2 - 1, iter_j, 0)

  wait_g(*A)
  if NCHUNK % 2:
    wait_s(idxvA, stageA, ssemA)
    compute(*CA)
  wait_ids(srcvB, dstvB, idsemB)
  wait_s(idxvA, stageA, ssemA)
  wait_s(idxvB, stageB, ssemB)


def _sc_edge_scratch(C):
  return [
      pltpu.VMEM((EB,), jnp.int32),
      pltpu.VMEM((EB,), jnp.int32),
      pltpu.VMEM((EB,), jnp.int32),
      pltpu.VMEM((EB,), jnp.int32),
      pltpu.VMEM((EB, C), jnp.float32),
      pltpu.VMEM((EB, C), jnp.float32),
      pltpu.VMEM((EB, C), jnp.float32),
      pltpu.VMEM((EB, C), jnp.float32),
      pltpu.VMEM((EB, C), jnp.float32),
      pltpu.VMEM((EB, C), jnp.float32),
      pltpu.VMEM((EB,), jnp.int32),
      pltpu.VMEM((EB,), jnp.int32),
      pltpu.VMEM((EB, C), jnp.float32),
      pltpu.VMEM((EB, C), jnp.float32),
      pltpu.VMEM((EB,), jnp.float32),
      pltpu.VMEM((EB,), jnp.float32),
      pltpu.SemaphoreType.DMA,
      pltpu.SemaphoreType.DMA,
      pltpu.SemaphoreType.DMA,
      pltpu.SemaphoreType.DMA,
      pltpu.SemaphoreType.DMA,
      pltpu.SemaphoreType.DMA,
  ]


def _sc_layer(N, E, C):
  NHALF = N // NC
  TPE = E // NS
  NCHUNK = TPE // EB
  NCHN = NHALF // CH
  SCALE = 1.0 / math.sqrt(float(C))
  mesh = plsc.VectorSubcoreMesh(core_axis_name="c", subcore_axis_name="s")

  @functools.partial(
      pl.kernel,
      out_type=jax.ShapeDtypeStruct((N, C), jnp.float32),
      mesh=mesh,
      compiler_params=_SC_PARAMS,
      scratch_types=_sc_edge_scratch(C) + [
          pltpu.VMEM((CH, C), jnp.float32),
          pltpu.VMEM((CH, C), jnp.float32),
          pltpu.VMEM((CH,), jnp.float32),
          pltpu.VMEM((CH, C), jnp.float32),
          pltpu.VMEM((CH, C), jnp.float32),
          pltpu.VMEM_SHARED((NHALF, C), jnp.float32),
          pltpu.VMEM_SHARED((NHALF,), jnp.float32),
      ],
  )
  def k(qt_ref, kt_ref, vt_ref, xs_ref, src_ref, dst_ref, h_ref,
        *scr):
    bufs = scr[:22]
    (zbuf, accv, denv, xsv, hv, acc, den) = scr[22:]
    c = lax.axis_index("c")
    t = lax.axis_index("s")
    lo = c * NHALF
    hi = lo + NHALF
    z16 = jnp.zeros((16,), jnp.float32)
    iota16 = lax.iota(jnp.int32, 16)

    def zrow(b, carry):
      for col0 in range(0, C, 16):
        zbuf[b, pl.ds(col0, 16)] = z16
      return carry
    lax.fori_loop(0, CH, zrow, 0)
    for g in range(CH // 16):
      denv[pl.ds(g * 16, 16)] = z16

    nz = (NCHN - t + NS - 1) // NS
    def zacc(j, carry):
      kk = t + j * NS
      pltpu.sync_copy(zbuf, acc.at[pl.ds(kk * CH, CH)])
      pltpu.sync_copy(denv, den.at[pl.ds(kk * CH, CH)])
      return carry
    lax.fori_loop(0, nz, zacc, 0)
    plsc.subcore_barrier()

    _edge_loop(C, lo, hi, NCHUNK, t * TPE, qt_ref, kt_ref, vt_ref,
               src_ref, dst_ref, bufs, acc, den, SCALE)
    plsc.subcore_barrier()

    nd = (NCHN - t + NS - 1) // NS
    def drain(j, carry):
      row0 = (t + j * NS) * CH
      pltpu.sync_copy(acc.at[pl.ds(row0, CH)], accv)
      pltpu.sync_copy(den.at[pl.ds(row0, CH)], denv)
      pltpu.sync_copy(xs_ref.at[pl.ds(lo + row0, CH)], xsv)
      def dgroup(g, cr):
        rows = iota16 + g * 16
        d16 = denv[pl.ds(g * 16, 16)]
        r = 1.0 / jnp.where(d16 > 0.0, d16, 1.0)
        col = jnp.zeros((16,), jnp.int32)
        for _ in range(C):
          num = plsc.load_gather(accv, [rows, col])
          xsc = plsc.load_gather(xsv, [rows, col])
          plsc.store_scatter(hv, [rows, col], jnp.maximum(num * r + xsc, 0.0))
          col = col + 1
        return cr
      lax.fori_loop(0, CH // 16, dgroup, 0)
      pltpu.sync_copy(hv, h_ref.at[pl.ds(lo + row0, CH)])
      return carry
    lax.fori_loop(0, nd, drain, 0)

  return k


def _sc_pool(N):
  NHALF = N // NC
  NCHN = NHALF // CH
  GPT = GG // NS
  mesh = plsc.VectorSubcoreMesh(core_axis_name="c", subcore_axis_name="s")

  @functools.partial(
      pl.kernel,
      out_type=jax.ShapeDtypeStruct((NC, GG, 40), jnp.float32),
      mesh=mesh,
      compiler_params=_SC_PARAMS,
      scratch_types=[
          pltpu.VMEM((CH, 32), jnp.float32),
          pltpu.VMEM((CH, 40), jnp.float32),
          pltpu.VMEM((GG // NS, 40), jnp.float32),
          pltpu.VMEM((CH,), jnp.int32),
          pltpu.VMEM_SHARED((GG, 40), jnp.float32),
      ],
  )
  def k(h_ref, batch_ref, out_ref, hvv, pstage, zpool, bidxv, pool):
    c = lax.axis_index("c")
    t = lax.axis_index("s")
    lo = c * NHALF
    z16 = jnp.zeros((16,), jnp.float32)
    cnt16 = jnp.where(lax.iota(jnp.int32, 16) == 8, 1.0, 0.0)

    def zrow(b, carry):
      pstage[b, pl.ds(0, 16)] = z16
      pstage[b, pl.ds(24, 16)] = cnt16
      return carry
    lax.fori_loop(0, CH, zrow, 0)
    def zprow(b, carry):
      zpool[b, pl.ds(0, 16)] = z16
      zpool[b, pl.ds(16, 16)] = z16
      zpool[b, pl.ds(24, 16)] = z16
      return carry
    lax.fori_loop(0, GPT, zprow, 0)
    pltpu.sync_copy(zpool, pool.at[pl.ds(t * GPT, GPT)])
    plsc.subcore_barrier()

    nd = (NCHN - t + NS - 1) // NS
    def pchunk(j, carry):
      row0 = (t + j * NS) * CH
      pltpu.sync_copy(h_ref.at[pl.ds(lo + row0, CH)], hvv)
      pltpu.sync_copy(batch_ref.at[pl.ds(lo + row0, CH)], bidxv)
      def node(nn, cr):
        pstage[nn, pl.ds(0, 16)] = hvv[nn, pl.ds(0, 16)]
        pstage[nn, pl.ds(16, 16)] = hvv[nn, pl.ds(16, 16)]
        return cr
      lax.fori_loop(0, CH, node, 0)
      pltpu.sync_copy(pstage, pool.at[bidxv], add=True)
      return carry
    lax.fori_loop(0, nd, pchunk, 0)
    plsc.subcore_barrier()

    pltpu.sync_copy(pool.at[pl.ds(t * GPT, GPT)],
                    out_ref.at[c, pl.ds(t * GPT, GPT)])

  return k


def _tc_proj1(N, R):
  def body(x_ref, wq_ref, bq_ref, wk_ref, bk_ref, wv_ref, bv_ref,
           ws_ref, bs_ref, qt_ref, kt_ref, vt_ref, xs_ref):
    xb = x_ref[...]
    qt_ref[...] = jnp.dot(xb, wq_ref[...], preferred_element_type=jnp.float32) + bq_ref[...]
    kt_ref[...] = jnp.dot(xb, wk_ref[...], preferred_element_type=jnp.float32) + bk_ref[...]
    vt_ref[...] = jnp.dot(xb, wv_ref[...], preferred_element_type=jnp.float32) + bv_ref[...]
    xs_ref[...] = jnp.dot(xb, ws_ref[...], preferred_element_type=jnp.float32) + bs_ref[...]

  grid = (N // R,)
  bw = pl.BlockSpec((9, 16), lambda i: (0, 0))
  bb = pl.BlockSpec((1, 16), lambda i: (0, 0))
  return pl.pallas_call(
      body,
      grid=grid,
      in_specs=[pl.BlockSpec((R, 9), lambda i: (i, 0)),
                bw, bb, bw, bb, bw, bb, bw, bb],
      out_specs=[pl.BlockSpec((R, 16), lambda i: (i, 0)),
                 pl.BlockSpec((R, 16), lambda i: (i, 0)),
                 pl.BlockSpec((R, 16), lambda i: (i, 0)),
                 pl.BlockSpec((R, 16), lambda i: (i, 0))],
      out_shape=[jax.ShapeDtypeStruct((N, 16), jnp.float32),
                 jax.ShapeDtypeStruct((N, 16), jnp.float32),
                 jax.ShapeDtypeStruct((N, 16), jnp.float32),
                 jax.ShapeDtypeStruct((N, 16), jnp.float32)],
  )


def _tc_proj2(N, R):
  def body(h_ref, wq_ref, bq_ref, wk_ref, bk_ref, wv_ref, bv_ref,
           ws_ref, bs_ref, qt_ref, kt_ref, vt_ref, xs_ref):
    hb = h_ref[...]
    qt_ref[...] = jnp.dot(hb, wq_ref[...], preferred_element_type=jnp.float32) + bq_ref[...]
    kt_ref[...] = jnp.dot(hb, wk_ref[...], preferred_element_type=jnp.float32) + bk_ref[...]
    vt_ref[...] = jnp.dot(hb, wv_ref[...], preferred_element_type=jnp.float32) + bv_ref[...]
    xs_ref[...] = jnp.dot(hb, ws_ref[...], preferred_element_type=jnp.float32) + bs_ref[...]

  grid = (N // R,)
  bw = pl.BlockSpec((16, 32), lambda i: (0, 0))
  bb = pl.BlockSpec((1, 32), lambda i: (0, 0))
  return pl.pallas_call(
      body,
      grid=grid,
      in_specs=[pl.BlockSpec((R, 16), lambda i: (i, 0)),
                bw, bb, bw, bb, bw, bb, bw, bb],
      out_specs=[pl.BlockSpec((R, 32), lambda i: (i, 0)),
                 pl.BlockSpec((R, 32), lambda i: (i, 0)),
                 pl.BlockSpec((R, 32), lambda i: (i, 0)),
                 pl.BlockSpec((R, 32), lambda i: (i, 0))],
      out_shape=[jax.ShapeDtypeStruct((N, 32), jnp.float32),
                 jax.ShapeDtypeStruct((N, 32), jnp.float32),
                 jax.ShapeDtypeStruct((N, 32), jnp.float32),
                 jax.ShapeDtypeStruct((N, 32), jnp.float32)],
  )


def _tc_head():
  def body(pp_ref, w1_ref, b1_ref, w2_ref, b2_ref, out_ref):
    p = pp_ref[0] + pp_ref[1]
    cnt = jnp.maximum(p[:, 32:33], 1.0)
    pooled = p[:, 0:32] / cnt
    h = jnp.maximum(
        jnp.dot(pooled, w1_ref[...], preferred_element_type=jnp.float32) + b1_ref[...], 0.0)
    o = jnp.dot(h, w2_ref[...], preferred_element_type=jnp.float32) + b2_ref[...]
    out_ref[...] = jax.nn.sigmoid(o)

  return pl.pallas_call(
      body,
      out_shape=jax.ShapeDtypeStruct((GG, 2), jnp.float32),
  )


def kernel(x, edge_index, batch, Wq1, bq1, Wk1, bk1, Wv1, bv1, Ws1, bs1,
           Wq2, bq2, Wk2, bk2, Wv2, bv2, Ws2, bs2, Wfc1, bfc1, Wfc2, bfc2):
  N = x.shape[0]
  E = edge_index.shape[1]
  assert N % (NC * CH) == 0 and E % (NS * EB) == 0
  src = edge_index[0]
  dst = edge_index[1]

  R = 2000
  qt1, kt1, vt1, xs1 = _tc_proj1(N, R)(
      x, Wq1, bq1.reshape(1, -1), Wk1, bk1.reshape(1, -1),
      Wv1, bv1.reshape(1, -1), Ws1, bs1.reshape(1, -1))
  h1 = _sc_layer(N, E, 16)(qt1, kt1, vt1, xs1, src, dst)
  qt2, kt2, vt2, xs2 = _tc_proj2(N, R)(
      h1, Wq2, bq2.reshape(1, -1), Wk2, bk2.reshape(1, -1),
      Wv2, bv2.reshape(1, -1), Ws2, bs2.reshape(1, -1))
  h2 = _sc_layer(N, E, 32)(qt2, kt2, vt2, xs2, src, dst)
  pp = _sc_pool(N)(h2, batch)
  return _tc_head()(pp, Wfc1, bfc1.reshape(1, -1), Wfc2, bfc2.reshape(1, -1))


# speedup vs baseline: 1.1547x; 1.1547x over previous
import functools
import math

import jax
import jax.numpy as jnp
from jax import lax
from jax.experimental import pallas as pl
from jax.experimental.pallas import tpu as pltpu
from jax.experimental.pallas import tpu_sc as plsc

NC = 2
NS = 16
GG = 1024

EB = 64
CH = 80

_SC_PARAMS = pltpu.CompilerParams(
    needs_layout_passes=False, use_tc_tiling_on_sc=False)


def _edge_compute(C, lo, hi, qrows, krows, vrows, dstv, idxv, stage,
                  denstage, SCALE):
  iota16 = lax.iota(jnp.int32, 16)

  def group(g, carry):
    s0 = g * 16
    rows = iota16 + s0
    accs = [jnp.zeros((16,), jnp.float32) for _ in range(4)]
    col = jnp.zeros((16,), jnp.int32)
    for u in range(C):
      accs[u % 4] += (plsc.load_gather(qrows, [rows, col]) *
                      plsc.load_gather(krows, [rows, col]))
      col = col + 1
    a = (accs[0] + accs[1]) + (accs[2] + accs[3])
    d16 = dstv[pl.ds(s0, 16)]
    m = (d16 >= lo) & (d16 < hi)
    e16 = jnp.where(m, jnp.exp(a * SCALE), 0.0)
    denstage[pl.ds(s0, 16)] = e16
    idxv[pl.ds(s0, 16)] = jnp.where(m, d16 - lo, d16 & 32767)
    col = jnp.zeros((16,), jnp.int32)
    for _ in range(C):
      vc = plsc.load_gather(vrows, [rows, col])
      plsc.store_scatter(stage, [rows, col], e16 * vc)
      col = col + 1
    return carry
  lax.fori_loop(0, EB // 16, group, 0)


def _edge_loop(C, lo, hi, NCHUNK, tbase, qt_ref, kt_ref, vt_ref,
               src_ref, dst_ref, bufs, acc, den, SCALE):
  (srcvA, dstvA, srcvB, dstvB, qrA, krA, vrA, qrB, krB, vrB,
   idxvA, idxvB, stageA, stageB, denstageA, denstageB,
   gsemA, gsemB, idsemA, idsemB) = bufs

  def issue_ids(base, srcv, dstv, sem):
    pltpu.async_copy(src_ref.at[pl.ds(base, EB)], srcv, sem)
    pltpu.async_copy(dst_ref.at[pl.ds(base, EB)], dstv, sem)

  def wait_ids(srcv, dstv, sem):
    pltpu.make_async_copy(src_ref.at[pl.ds(0, EB)], srcv, sem).wait()
    pltpu.make_async_copy(dst_ref.at[pl.ds(0, EB)], dstv, sem).wait()

  def issue_g(srcv, dstv, qr, kr, vr, sem):
    pltpu.async_copy(qt_ref.at[dstv], qr, sem)
    pltpu.async_copy(kt_ref.at[srcv], kr, sem)
    pltpu.async_copy(vt_ref.at[srcv], vr, sem)

  def wait_g(srcv, dstv, qr, kr, vr, sem):
    pltpu.make_async_copy(qt_ref.at[dstv], qr, sem).wait()
    pltpu.make_async_copy(kt_ref.at[srcv], kr, sem).wait()
    pltpu.make_async_copy(vt_ref.at[srcv], vr, sem).wait()

  def compute(srcv, dstv, qr, kr, vr, idxv, stage, denstage):
    _edge_compute(C, lo, hi, qr, kr, vr, dstv, idxv, stage, denstage, SCALE)
    pltpu.sync_copy(stage, acc.at[idxv], add=True)
    pltpu.sync_copy(denstage, den.at[idxv], add=True)

  A = (srcvA, dstvA, qrA, krA, vrA, gsemA)
  B = (srcvB, dstvB, qrB, krB, vrB, gsemB)
  CA = (srcvA, dstvA, qrA, krA, vrA, idxvA, stageA, denstageA)
  CB = (srcvB, dstvB, qrB, krB, vrB, idxvB, stageB, denstageB)

  issue_ids(tbase, srcvA, dstvA, idsemA)
  wait_ids(srcvA, dstvA, idsemA)
  issue_g(*A)
  issue_ids(tbase + EB, srcvB, dstvB, idsemB)

  def iter_j(j, carry):
    a = 2 * j
    b = a + 1
    wait_ids(srcvB, dstvB, idsemB)
    wait_g(*A)
    issue_g(*B)
    compute(*CA)
    nxt = jnp.minimum(a + 2, NCHUNK - 1)
    issue_ids(tbase + nxt * EB, srcvA, dstvA, idsemA)
    wait_g(*B)
    compute(*CB)
    wait_ids(srcvA, dstvA, idsemA)
    issue_g(*A)
    nxt2 = jnp.minimum(b + 2, NCHUNK - 1)
    issue_ids(tbase + nxt2 * EB, srcvB, dstvB, idsemB)
    return carry
  lax.fori_loop(0, NCHUNK // 2, iter_j, 0)

  wait_g(*A)
  if NCHUNK % 2:
    compute(*CA)
  wait_ids(srcvB, dstvB, idsemB)


def _sc_edge_scratch(C):
  return [
      pltpu.VMEM((EB,), jnp.int32),
      pltpu.VMEM((EB,), jnp.int32),
      pltpu.VMEM((EB,), jnp.int32),
      pltpu.VMEM((EB,), jnp.int32),
      pltpu.VMEM((EB, C), jnp.float32),
      pltpu.VMEM((EB, C), jnp.float32),
      pltpu.VMEM((EB, C), jnp.float32),
      pltpu.VMEM((EB, C), jnp.float32),
      pltpu.VMEM((EB, C), jnp.float32),
      pltpu.VMEM((EB, C), jnp.float32),
      pltpu.VMEM((EB,), jnp.int32),
      pltpu.VMEM((EB,), jnp.int32),
      pltpu.VMEM((EB, C), jnp.float32),
      pltpu.VMEM((EB, C), jnp.float32),
      pltpu.VMEM((EB,), jnp.float32),
      pltpu.VMEM((EB,), jnp.float32),
      pltpu.SemaphoreType.DMA,
      pltpu.SemaphoreType.DMA,
      pltpu.SemaphoreType.DMA,
      pltpu.SemaphoreType.DMA,
  ]


def _sc_layer(N, E, C):
  NHALF = N // NC
  TPE = E // NS
  NCHUNK = TPE // EB
  NCHN = NHALF // CH
  SCALE = 1.0 / math.sqrt(float(C))
  mesh = plsc.VectorSubcoreMesh(core_axis_name="c", subcore_axis_name="s")

  @functools.partial(
      pl.kernel,
      out_type=jax.ShapeDtypeStruct((N, C), jnp.float32),
      mesh=mesh,
      compiler_params=_SC_PARAMS,
      scratch_types=_sc_edge_scratch(C) + [
          pltpu.VMEM((CH, C), jnp.float32),
          pltpu.VMEM((CH, C), jnp.float32),
          pltpu.VMEM((CH,), jnp.float32),
          pltpu.VMEM((CH, C), jnp.float32),
          pltpu.VMEM((CH, C), jnp.float32),
          pltpu.VMEM_SHARED((NHALF, C), jnp.float32),
          pltpu.VMEM_SHARED((NHALF,), jnp.float32),
      ],
  )
  def k(qt_ref, kt_ref, vt_ref, xs_ref, src_ref, dst_ref, h_ref,
        *scr):
    bufs = scr[:20]
    (zbuf, accv, denv, xsv, hv, acc, den) = scr[20:]
    c = lax.axis_index("c")
    t = lax.axis_index("s")
    lo = c * NHALF
    hi = lo + NHALF
    z16 = jnp.zeros((16,), jnp.float32)
    iota16 = lax.iota(jnp.int32, 16)

    def zrow(b, carry):
      for col0 in range(0, C, 16):
        zbuf[b, pl.ds(col0, 16)] = z16
      return carry
    lax.fori_loop(0, CH, zrow, 0)
    for g in range(CH // 16):
      denv[pl.ds(g * 16, 16)] = z16

    nz = (NCHN - t + NS - 1) // NS
    def zacc(j, carry):
      kk = t + j * NS
      pltpu.sync_copy(zbuf, acc.at[pl.ds(kk * CH, CH)])
      pltpu.sync_copy(denv, den.at[pl.ds(kk * CH, CH)])
      return carry
    lax.fori_loop(0, nz, zacc, 0)
    plsc.subcore_barrier()

    _edge_loop(C, lo, hi, NCHUNK, t * TPE, qt_ref, kt_ref, vt_ref,
               src_ref, dst_ref, bufs, acc, den, SCALE)
    plsc.subcore_barrier()

    nd = (NCHN - t + NS - 1) // NS
    def drain(j, carry):
      row0 = (t + j * NS) * CH
      pltpu.sync_copy(acc.at[pl.ds(row0, CH)], accv)
      pltpu.sync_copy(den.at[pl.ds(row0, CH)], denv)
      pltpu.sync_copy(xs_ref.at[pl.ds(lo + row0, CH)], xsv)
      def dgroup(g, cr):
        rows = iota16 + g * 16
        d16 = denv[pl.ds(g * 16, 16)]
        r = 1.0 / jnp.where(d16 > 0.0, d16, 1.0)
        col = jnp.zeros((16,), jnp.int32)
        for _ in range(C):
          num = plsc.load_gather(accv, [rows, col])
          xsc = plsc.load_gather(xsv, [rows, col])
          plsc.store_scatter(hv, [rows, col], jnp.maximum(num * r + xsc, 0.0))
          col = col + 1
        return cr
      lax.fori_loop(0, CH // 16, dgroup, 0)
      pltpu.sync_copy(hv, h_ref.at[pl.ds(lo + row0, CH)])
      return carry
    lax.fori_loop(0, nd, drain, 0)

  return k


def _sc_pool(N):
  NHALF = N // NC
  NCHN = NHALF // CH
  GPT = GG // NS
  mesh = plsc.VectorSubcoreMesh(core_axis_name="c", subcore_axis_name="s")

  @functools.partial(
      pl.kernel,
      out_type=jax.ShapeDtypeStruct((NC, GG, 40), jnp.float32),
      mesh=mesh,
      compiler_params=_SC_PARAMS,
      scratch_types=[
          pltpu.VMEM((CH, 32), jnp.float32),
          pltpu.VMEM((CH, 40), jnp.float32),
          pltpu.VMEM((GG // NS, 40), jnp.float32),
          pltpu.VMEM((CH,), jnp.int32),
          pltpu.VMEM_SHARED((GG, 40), jnp.float32),
      ],
  )
  def k(h_ref, batch_ref, out_ref, hvv, pstage, zpool, bidxv, pool):
    c = lax.axis_index("c")
    t = lax.axis_index("s")
    lo = c * NHALF
    z16 = jnp.zeros((16,), jnp.float32)
    cnt16 = jnp.where(lax.iota(jnp.int32, 16) == 8, 1.0, 0.0)

    def zrow(b, carry):
      pstage[b, pl.ds(0, 16)] = z16
      pstage[b, pl.ds(24, 16)] = cnt16
      return carry
    lax.fori_loop(0, CH, zrow, 0)
    def zprow(b, carry):
      zpool[b, pl.ds(0, 16)] = z16
      zpool[b, pl.ds(16, 16)] = z16
      zpool[b, pl.ds(24, 16)] = z16
      return carry
    lax.fori_loop(0, GPT, zprow, 0)
    pltpu.sync_copy(zpool, pool.at[pl.ds(t * GPT, GPT)])
    plsc.subcore_barrier()

    nd = (NCHN - t + NS - 1) // NS
    def pchunk(j, carry):
      row0 = (t + j * NS) * CH
      pltpu.sync_copy(h_ref.at[pl.ds(lo + row0, CH)], hvv)
      pltpu.sync_copy(batch_ref.at[pl.ds(lo + row0, CH)], bidxv)
      def node(nn, cr):
        pstage[nn, pl.ds(0, 16)] = hvv[nn, pl.ds(0, 16)]
        pstage[nn, pl.ds(16, 16)] = hvv[nn, pl.ds(16, 16)]
        return cr
      lax.fori_loop(0, CH, node, 0)
      pltpu.sync_copy(pstage, pool.at[bidxv], add=True)
      return carry
    lax.fori_loop(0, nd, pchunk, 0)
    plsc.subcore_barrier()

    pltpu.sync_copy(pool.at[pl.ds(t * GPT, GPT)],
                    out_ref.at[c, pl.ds(t * GPT, GPT)])

  return k


def _tc_proj1(N, R):
  def body(x_ref, wq_ref, bq_ref, wk_ref, bk_ref, wv_ref, bv_ref,
           ws_ref, bs_ref, qt_ref, kt_ref, vt_ref, xs_ref):
    xb = x_ref[...]
    qt_ref[...] = jnp.dot(xb, wq_ref[...], preferred_element_type=jnp.float32) + bq_ref[...]
    kt_ref[...] = jnp.dot(xb, wk_ref[...], preferred_element_type=jnp.float32) + bk_ref[...]
    vt_ref[...] = jnp.dot(xb, wv_ref[...], preferred_element_type=jnp.float32) + bv_ref[...]
    xs_ref[...] = jnp.dot(xb, ws_ref[...], preferred_element_type=jnp.float32) + bs_ref[...]

  grid = (N // R,)
  bw = pl.BlockSpec((9, 16), lambda i: (0, 0))
  bb = pl.BlockSpec((1, 16), lambda i: (0, 0))
  return pl.pallas_call(
      body,
      grid=grid,
      in_specs=[pl.BlockSpec((R, 9), lambda i: (i, 0)),
                bw, bb, bw, bb, bw, bb, bw, bb],
      out_specs=[pl.BlockSpec((R, 16), lambda i: (i, 0)),
                 pl.BlockSpec((R, 16), lambda i: (i, 0)),
                 pl.BlockSpec((R, 16), lambda i: (i, 0)),
                 pl.BlockSpec((R, 16), lambda i: (i, 0))],
      out_shape=[jax.ShapeDtypeStruct((N, 16), jnp.float32),
                 jax.ShapeDtypeStruct((N, 16), jnp.float32),
                 jax.ShapeDtypeStruct((N, 16), jnp.float32),
                 jax.ShapeDtypeStruct((N, 16), jnp.float32)],
  )


def _tc_proj2(N, R):
  def body(h_ref, wq_ref, bq_ref, wk_ref, bk_ref, wv_ref, bv_ref,
           ws_ref, bs_ref, qt_ref, kt_ref, vt_ref, xs_ref):
    hb = h_ref[...]
    qt_ref[...] = jnp.dot(hb, wq_ref[...], preferred_element_type=jnp.float32) + bq_ref[...]
    kt_ref[...] = jnp.dot(hb, wk_ref[...], preferred_element_type=jnp.float32) + bk_ref[...]
    vt_ref[...] = jnp.dot(hb, wv_ref[...], preferred_element_type=jnp.float32) + bv_ref[...]
    xs_ref[...] = jnp.dot(hb, ws_ref[...], preferred_element_type=jnp.float32) + bs_ref[...]

  grid = (N // R,)
  bw = pl.BlockSpec((16, 32), lambda i: (0, 0))
  bb = pl.BlockSpec((1, 32), lambda i: (0, 0))
  return pl.pallas_call(
      body,
      grid=grid,
      in_specs=[pl.BlockSpec((R, 16), lambda i: (i, 0)),
                bw, bb, bw, bb, bw, bb, bw, bb],
      out_specs=[pl.BlockSpec((R, 32), lambda i: (i, 0)),
                 pl.BlockSpec((R, 32), lambda i: (i, 0)),
                 pl.BlockSpec((R, 32), lambda i: (i, 0)),
                 pl.BlockSpec((R, 32), lambda i: (i, 0))],
      out_shape=[jax.ShapeDtypeStruct((N, 32), jnp.float32),
                 jax.ShapeDtypeStruct((N, 32), jnp.float32),
                 jax.ShapeDtypeStruct((N, 32), jnp.float32),
                 jax.ShapeDtypeStruct((N, 32), jnp.float32)],
  )


def _tc_head():
  def body(pp_ref, w1_ref, b1_ref, w2_ref, b2_ref, out_ref):
    p = pp_ref[0] + pp_ref[1]
    cnt = jnp.maximum(p[:, 32:33], 1.0)
    pooled = p[:, 0:32] / cnt
    h = jnp.maximum(
        jnp.dot(pooled, w1_ref[...], preferred_element_type=jnp.float32) + b1_ref[...], 0.0)
    o = jnp.dot(h, w2_ref[...], preferred_element_type=jnp.float32) + b2_ref[...]
    out_ref[...] = jax.nn.sigmoid(o)

  return pl.pallas_call(
      body,
      out_shape=jax.ShapeDtypeStruct((GG, 2), jnp.float32),
  )


def kernel(x, edge_index, batch, Wq1, bq1, Wk1, bk1, Wv1, bv1, Ws1, bs1,
           Wq2, bq2, Wk2, bk2, Wv2, bv2, Ws2, bs2, Wfc1, bfc1, Wfc2, bfc2):
  N = x.shape[0]
  E = edge_index.shape[1]
  assert N % (NC * CH) == 0 and E % (NS * EB) == 0
  src = edge_index[0]
  dst = edge_index[1]

  R = 2000
  qt1, kt1, vt1, xs1 = _tc_proj1(N, R)(
      x, Wq1, bq1.reshape(1, -1), Wk1, bk1.reshape(1, -1),
      Wv1, bv1.reshape(1, -1), Ws1, bs1.reshape(1, -1))
  h1 = _sc_layer(N, E, 16)(qt1, kt1, vt1, xs1, src, dst)
  qt2, kt2, vt2, xs2 = _tc_proj2(N, R)(
      h1, Wq2, bq2.reshape(1, -1), Wk2, bk2.reshape(1, -1),
      Wv2, bv2.reshape(1, -1), Ws2, bs2.reshape(1, -1))
  h2 = _sc_layer(N, E, 32)(qt2, kt2, vt2, xs2, src, dst)
  pp = _sc_pool(N)(h2, batch)
  return _tc_head()(pp, Wfc1, bfc1.reshape(1, -1), Wfc2, bfc2.reshape(1, -1))
